# Initial kernel scaffold; baseline (speedup 1.0000x reference)
import functools

import jax
import jax.numpy as jnp
from jax import lax
from jax.experimental import pallas as pl
from jax.experimental.pallas import tpu as pltpu
from jax.experimental.pallas import tpu_sc as plsc

B = 4096 * 200
F = 32
NC = 2
NS = 16
NW = NC * NS
ROWS_PER_W = B // NW
CHUNK = 512
SUB = 128
NSUB = CHUNK // SUB
NCHUNK = ROWS_PER_W // CHUNK
NBUF = 2
NSTEP = NCHUNK // NBUF
GROUPS = CHUNK // 16


def _gather_chunk(x_hbm, tab_hbm, base, idx, rows, gsem):
    pltpu.sync_copy(x_hbm.at[pl.ds(base, CHUNK)], idx)
    cps = []
    for j in range(NSUB):
        cps.append(pltpu.async_copy(
            tab_hbm.at[idx.at[pl.ds(j * SUB, SUB)]],
            rows.at[pl.ds(j * SUB, SUB)],
            gsem))
    return cps


def _mask_chunk(idx, rows):
    def cnt(k, acc):
        v = idx[pl.ds(pl.multiple_of(k * 16, 16), 16)]
        return acc + jnp.where(v <= 0, 1, 0).astype(jnp.int32)

    acc = lax.fori_loop(0, GROUPS, cnt, jnp.zeros((16,), jnp.int32))
    nz = jnp.sum(acc)

    @pl.when(nz > 0)
    def _():
        zeros = jnp.zeros((16,), jnp.float32)

        def fix(k, carry):
            v = idx[pl.ds(pl.multiple_of(k * 16, 16), 16)]
            m = v <= 0
            rid = lax.iota(jnp.int32, 16) + k * 16
            for c in range(F):
                plsc.store_scatter(
                    rows, [rid, jnp.full((16,), c, jnp.int32)],
                    zeros, mask=m)
            return carry

        lax.fori_loop(0, GROUPS, fix, 0)


@functools.partial(
    pl.kernel,
    out_type=jax.ShapeDtypeStruct((B, F), jnp.float32),
    mesh=plsc.VectorSubcoreMesh(core_axis_name="c", subcore_axis_name="s"),
    scratch_types=[
        pltpu.VMEM((CHUNK,), jnp.int32),
        pltpu.VMEM((CHUNK,), jnp.int32),
        pltpu.VMEM((CHUNK, F), jnp.float32),
        pltpu.VMEM((CHUNK, F), jnp.float32),
        pltpu.SemaphoreType.DMA,
        pltpu.SemaphoreType.DMA,
        pltpu.SemaphoreType.DMA,
        pltpu.SemaphoreType.DMA,
    ],
)
def _embed(x_hbm, tab_hbm, out_hbm,
           idx0, idx1, rows0, rows1, gsem0, gsem1, wsem0, wsem1):
    wid = lax.axis_index("s") * NC + lax.axis_index("c")
    wbase = wid * ROWS_PER_W
    bufs = ((idx0, rows0, gsem0, wsem0), (idx1, rows1, gsem1, wsem1))

    def step(i, carry):
        gcps = []
        for b in range(NBUF):
            idx, rows, gsem, _ = bufs[b]
            base = wbase + (i * NBUF + b) * CHUNK
            gcps.append(_gather_chunk(x_hbm, tab_hbm, base, idx, rows, gsem))
        wcps = []
        for b in range(NBUF):
            idx, rows, _, wsem = bufs[b]
            base = wbase + (i * NBUF + b) * CHUNK
            for cp in gcps[b]:
                cp.wait()
            _mask_chunk(idx, rows)
            wcps.append(pltpu.async_copy(
                rows, out_hbm.at[pl.ds(base, CHUNK)], wsem))
        for cp in wcps:
            cp.wait()
        return carry

    lax.fori_loop(0, NSTEP, step, 0)


def kernel(x, table):
    x1 = x.reshape(B).astype(jnp.int32)
    out = _embed(x1, table)
    return out.reshape(x.shape[0], x.shape[1], F)


# speedup vs baseline: 1.4804x; 1.4804x over previous
import functools

import jax
import jax.numpy as jnp
from jax import lax
from jax.experimental import pallas as pl
from jax.experimental.pallas import tpu as pltpu
from jax.experimental.pallas import tpu_sc as plsc

B = 4096 * 200
F = 32
NC = 2
NS = 16
NW = NC * NS
ROWS_PER_W = B // NW
CHUNK = 512
SUB = 128
NSUB = CHUNK // SUB
NCHUNK = ROWS_PER_W // CHUNK
NBUF = 2
NSTEP = NCHUNK // NBUF
GROUPS = CHUNK // 16


def _gather_chunk(x_hbm, tab_hbm, base, idx, rows, gsem):
    pltpu.sync_copy(x_hbm.at[pl.ds(base, CHUNK)], idx)
    cps = []
    for j in range(NSUB):
        cps.append(pltpu.async_copy(
            tab_hbm.at[idx.at[pl.ds(j * SUB, SUB)]],
            rows.at[pl.ds(j * SUB, SUB)],
            gsem))
    return cps


def _mask_chunk(idx, rows):
    def cnt(k, acc):
        v = idx[pl.ds(pl.multiple_of(k * 16, 16), 16)]
        return acc + jnp.where(v <= 0, 1, 0).astype(jnp.int32)

    acc = lax.fori_loop(0, GROUPS, cnt, jnp.zeros((16,), jnp.int32))
    nz = plsc.all_reduce_population_count(acc > 0)[0]

    @pl.when(nz > 0)
    def _():
        zeros = jnp.zeros((16,), jnp.float32)

        def fix(k, carry):
            v = idx[pl.ds(pl.multiple_of(k * 16, 16), 16)]
            m = v <= 0
            rid = lax.iota(jnp.int32, 16) + k * 16
            for c in range(F):
                plsc.store_scatter(
                    rows, [rid, jnp.full((16,), c, jnp.int32)],
                    zeros, mask=m)
            return carry

        lax.fori_loop(0, GROUPS, fix, 0)


@functools.partial(
    pl.kernel,
    out_type=jax.ShapeDtypeStruct((B, F), jnp.float32),
    mesh=plsc.VectorSubcoreMesh(core_axis_name="c", subcore_axis_name="s"),
    compiler_params=pltpu.CompilerParams(
        needs_layout_passes=False, use_tc_tiling_on_sc=False),
    scratch_types=[
        pltpu.VMEM((CHUNK,), jnp.int32),
        pltpu.VMEM((CHUNK,), jnp.int32),
        pltpu.VMEM((CHUNK, F), jnp.float32),
        pltpu.VMEM((CHUNK, F), jnp.float32),
        pltpu.SemaphoreType.DMA,
        pltpu.SemaphoreType.DMA,
        pltpu.SemaphoreType.DMA,
        pltpu.SemaphoreType.DMA,
    ],
)
def _embed(x_hbm, tab_hbm, out_hbm,
           idx0, idx1, rows0, rows1,
           gsem0, gsem1, wsem0, wsem1):
    wid = lax.axis_index("s") * NC + lax.axis_index("c")
    wbase = wid * ROWS_PER_W
    bufs = ((idx0, rows0, gsem0, wsem0), (idx1, rows1, gsem1, wsem1))

    def step(i, carry):
        gcps = []
        for b in range(NBUF):
            idx, rows, gsem, _ = bufs[b]
            base = wbase + (i * NBUF + b) * CHUNK
            gcps.append(_gather_chunk(x_hbm, tab_hbm, base, idx, rows, gsem))
        wcps = []
        for b in range(NBUF):
            idx, rows, _, wsem = bufs[b]
            base = wbase + (i * NBUF + b) * CHUNK
            for cp in gcps[b]:
                cp.wait()
            _mask_chunk(idx, rows)
            wcps.append(pltpu.async_copy(
                rows, out_hbm.at[pl.ds(base, CHUNK)], wsem))
        for cp in wcps:
            cp.wait()
        return carry

    lax.fori_loop(0, NSTEP, step, 0)


def kernel(x, table):
    x1 = x.reshape(B).astype(jnp.int32)
    out = _embed(x1, table)
    return out.reshape(x.shape[0], x.shape[1], F)


# speedup vs baseline: 1.5141x; 1.0228x over previous
import functools

import jax
import jax.numpy as jnp
from jax import lax
from jax.experimental import pallas as pl
from jax.experimental.pallas import tpu as pltpu
from jax.experimental.pallas import tpu_sc as plsc

B = 4096 * 200
F = 32
NC = 2
NS = 16
NW = NC * NS
ROWS_PER_W = B // NW
CHUNK = 512
SUB = 128
NSUB = CHUNK // SUB
NCHUNK = ROWS_PER_W // CHUNK
NBUF = 5
DEPTH = NBUF - 1
NSTEP = NCHUNK // NBUF
GROUPS = CHUNK // 16


def _fire_gather(tab_hbm, idx_all, rows, gsem, g):
    for j in range(NSUB):
        off = pl.multiple_of(g * CHUNK + j * SUB, SUB)
        pltpu.async_copy(
            tab_hbm.at[idx_all.at[pl.ds(off, SUB)]],
            rows.at[pl.ds(j * SUB, SUB)],
            gsem)


def _wait_gather(tab_hbm, idx_all, rows, gsem):
    for j in range(NSUB):
        pltpu.make_async_copy(
            tab_hbm.at[idx_all.at[pl.ds(j * SUB, SUB)]],
            rows.at[pl.ds(j * SUB, SUB)],
            gsem).wait()


def _wait_wb(out_hbm, rows, wsem):
    pltpu.make_async_copy(rows, out_hbm.at[pl.ds(0, CHUNK)], wsem).wait()


def _mask_chunk(idx_all, rows, g):
    def cnt(k, acc):
        off = pl.multiple_of(g * CHUNK + k * 16, 16)
        v = idx_all[pl.ds(off, 16)]
        return acc + jnp.where(v <= 0, 1, 0).astype(jnp.int32)

    acc = lax.fori_loop(0, GROUPS, cnt, jnp.zeros((16,), jnp.int32))
    nz = plsc.all_reduce_population_count(acc > 0)[0]

    @pl.when(nz > 0)
    def _():
        zeros = jnp.zeros((16,), jnp.float32)

        def fix(k, carry):
            off = pl.multiple_of(g * CHUNK + k * 16, 16)
            v = idx_all[pl.ds(off, 16)]
            m = v <= 0
            rid = lax.iota(jnp.int32, 16) + k * 16
            for c in range(F):
                plsc.store_scatter(
                    rows, [rid, jnp.full((16,), c, jnp.int32)],
                    zeros, mask=m)
            return carry

        lax.fori_loop(0, GROUPS, fix, 0)


@functools.partial(
    pl.kernel,
    out_type=jax.ShapeDtypeStruct((B, F), jnp.float32),
    mesh=plsc.VectorSubcoreMesh(core_axis_name="c", subcore_axis_name="s"),
    compiler_params=pltpu.CompilerParams(
        needs_layout_passes=False, use_tc_tiling_on_sc=False),
    scratch_types=[
        pltpu.VMEM((ROWS_PER_W,), jnp.int32),
        [pltpu.VMEM((CHUNK, F), jnp.float32) for _ in range(NBUF)],
        [pltpu.SemaphoreType.DMA for _ in range(NBUF)],
        [pltpu.SemaphoreType.DMA for _ in range(NBUF)],
    ],
)
def _embed(x_hbm, tab_hbm, out_hbm, idx_all, rows, gsem, wsem):
    wid = lax.axis_index("s") * NC + lax.axis_index("c")
    wbase = wid * ROWS_PER_W

    pltpu.sync_copy(x_hbm.at[pl.ds(wbase, ROWS_PER_W)], idx_all)

    for g in range(DEPTH):
        _fire_gather(tab_hbm, idx_all, rows[g], gsem[g], g)

    def step(i, carry):
        for b in range(NBUF):
            g = i * NBUF + b
            _wait_gather(tab_hbm, idx_all, rows[b], gsem[b])
            _mask_chunk(idx_all, rows[b], g)
            pltpu.async_copy(
                rows[b], out_hbm.at[pl.ds(wbase + g * CHUNK, CHUNK)],
                wsem[b])
            nb = (b + DEPTH) % NBUF
            if b == 0:
                @pl.when(i > 0)
                def _():
                    _wait_wb(out_hbm, rows[nb], wsem[nb])
                _fire_gather(tab_hbm, idx_all, rows[nb], gsem[nb], g + DEPTH)
            else:
                @pl.when(i < NSTEP - 1)
                def _():
                    _wait_wb(out_hbm, rows[nb], wsem[nb])
                    _fire_gather(tab_hbm, idx_all, rows[nb], gsem[nb],
                                 g + DEPTH)
        return carry

    lax.fori_loop(0, NSTEP, step, 0)

    for b in range(NBUF):
        _wait_wb(out_hbm, rows[b], wsem[b])


def kernel(x, table):
    x1 = x.reshape(B).astype(jnp.int32)
    out = _embed(x1, table)
    return out.reshape(x.shape[0], x.shape[1], F)


# speedup vs baseline: 1.5142x; 1.0000x over previous
import functools

import jax
import jax.numpy as jnp
from jax import lax
from jax.experimental import pallas as pl
from jax.experimental.pallas import tpu as pltpu
from jax.experimental.pallas import tpu_sc as plsc

BATCH = 4096
HIST = 200
F = 32
B = BATCH * HIST
NC = 2
NS = 16
NW = NC * NS
B_PER_W = BATCH // NW
ROWS_PER_W = B_PER_W * HIST
NBUF = 8
DEPTH = NBUF - 1
NSTEP = B_PER_W // NBUF
GROUP_OFFS = tuple(range(0, 192, 16)) + (184,)
STREAMS = ((0, 128), (128, 72))


def _fire_gather(tab_hbm, idx_all, rows, gsem, k):
    for (o, n) in STREAMS:
        off = pl.multiple_of(k * HIST + o, 8)
        pltpu.async_copy(
            tab_hbm.at[idx_all.at[pl.ds(off, n)]],
            rows.at[pl.ds(o, n)],
            gsem)


def _wait_gather(tab_hbm, idx_all, rows, gsem):
    for (o, n) in STREAMS:
        pltpu.make_async_copy(
            tab_hbm.at[idx_all.at[pl.ds(o, n)]],
            rows.at[pl.ds(o, n)],
            gsem).wait()


def _wait_wb(out_hbm, rows, wsem, b):
    pltpu.make_async_copy(rows, out_hbm.at[b], wsem).wait()


def _mask_chunk(idx_all, rows, k):
    def cnt(g, acc):
        off = pl.multiple_of(k * HIST + jnp.minimum(g * 16, 184), 8)
        v = idx_all[pl.ds(off, 16)]
        return acc + jnp.where(v <= 0, 1, 0).astype(jnp.int32)

    acc = lax.fori_loop(0, len(GROUP_OFFS), cnt, jnp.zeros((16,), jnp.int32))
    nz = plsc.all_reduce_population_count(acc > 0)[0]

    @pl.when(nz > 0)
    def _():
        zeros = jnp.zeros((16,), jnp.float32)

        def fix(g, carry):
            local = jnp.minimum(g * 16, 184)
            off = pl.multiple_of(k * HIST + local, 8)
            v = idx_all[pl.ds(off, 16)]
            m = v <= 0
            rid = lax.iota(jnp.int32, 16) + local
            for c in range(F):
                plsc.store_scatter(
                    rows, [rid, jnp.full((16,), c, jnp.int32)],
                    zeros, mask=m)
            return carry

        lax.fori_loop(0, len(GROUP_OFFS), fix, 0)


@functools.partial(
    pl.kernel,
    out_type=jax.ShapeDtypeStruct((BATCH, HIST, F), jnp.float32),
    mesh=plsc.VectorSubcoreMesh(core_axis_name="c", subcore_axis_name="s"),
    compiler_params=pltpu.CompilerParams(
        needs_layout_passes=False, use_tc_tiling_on_sc=False),
    scratch_types=[
        pltpu.VMEM((ROWS_PER_W,), jnp.int32),
        [pltpu.VMEM((HIST, F), jnp.float32) for _ in range(NBUF)],
        [pltpu.SemaphoreType.DMA for _ in range(NBUF)],
        [pltpu.SemaphoreType.DMA for _ in range(NBUF)],
    ],
)
def _embed(x_hbm, tab_hbm, out_hbm, idx_all, rows, gsem, wsem):
    wid = lax.axis_index("s") * NC + lax.axis_index("c")
    bbase = wid * B_PER_W

    pltpu.sync_copy(x_hbm.at[pl.ds(wid * ROWS_PER_W, ROWS_PER_W)], idx_all)

    for k in range(DEPTH):
        _fire_gather(tab_hbm, idx_all, rows[k], gsem[k], k)

    def step(i, carry):
        for b in range(NBUF):
            k = i * NBUF + b
            _wait_gather(tab_hbm, idx_all, rows[b], gsem[b])
            _mask_chunk(idx_all, rows[b], k)
            pltpu.async_copy(rows[b], out_hbm.at[bbase + k], wsem[b])
            nb = (b + DEPTH) % NBUF
            if b == 0:
                @pl.when(i > 0)
                def _():
                    _wait_wb(out_hbm, rows[nb], wsem[nb], bbase)
                _fire_gather(tab_hbm, idx_all, rows[nb], gsem[nb], k + DEPTH)
            else:
                @pl.when(i < NSTEP - 1)
                def _():
                    _wait_wb(out_hbm, rows[nb], wsem[nb], bbase)
                    _fire_gather(tab_hbm, idx_all, rows[nb], gsem[nb],
                                 k + DEPTH)
        return carry

    lax.fori_loop(0, NSTEP, step, 0)

    for b in range(NBUF):
        _wait_wb(out_hbm, rows[b], wsem[b], bbase)


def kernel(x, table):
    x1 = x.reshape(B).astype(jnp.int32)
    return _embed(x1, table)
